# baseline (speedup 1.0000x reference)
import functools

import jax
import jax.numpy as jnp
from jax import lax
from jax.experimental import pallas as pl
from jax.experimental.pallas import tpu as pltpu
from jax.experimental.pallas import tpu_sc as plsc

N = 10000
D = 256
E = 160000

NC = 2
NS = 16
NW = NC * NS
L = 16
HW = D // 2

OWN = 320
S = 800
NSTRIP = E // S
FU = 5
CAPF = 8192
ACC_R = 321
DUMMY = OWN
FG = D // L
BR = 100
NB = N // BR
BW = BR * HW


def _sc_agg_body(x_hbm, src_hbm, dst_hbm, agg_hbm,
                 srcv, dstv, srcv1, dstv1, srcf, dstf, mbs, mbd,
                 xblk0, xblk1, acc, bsem0, bsem1, esem0, esem1):
    c = lax.axis_index("c")
    s = lax.axis_index("s")
    wid = c * NS + s
    lo = wid * OWN
    lo_v = jnp.full((L,), lo, jnp.int32)
    own_u32 = jnp.full((L,), OWN, jnp.uint32)
    br_u32 = jnp.full((L,), BR, jnp.uint32)
    ones_v = jnp.ones((L,), jnp.int32)
    zeros_v = jnp.zeros((L,), jnp.int32)
    trash_v = jnp.full((L,), CAPF - 1, jnp.int32)
    mtrash_v = jnp.full((L,), 31, jnp.int32)
    dummy_v = jnp.full((L,), DUMMY, jnp.int32)
    iota_v = lax.iota(jnp.int32, L)
    last_v = jnp.full((L,), L - 1, jnp.int32)
    zero16f = jnp.zeros((L,), jnp.float32)
    pvecs = [iota_v + h * L for h in range(FG // 2)]
    favecs = [iota_v * 2 + h * 2 * L for h in range(FG // 2)]
    fbvecs = [favecs[h] + 1 for h in range(FG // 2)]

    def _zacc(i, _):
        plsc.store_scatter(acc, [jnp.full((L,), i // FG, jnp.int32),
                                 iota_v + (i % FG) * L], zero16f)
        return 0
    lax.fori_loop(0, ACC_R * FG, _zacc, 0)

    def _filt_from(bs, bd, i, cnt_v1):
        for u in range(FU):
            g = i * FU + u
            d = bd[pl.ds(g * L, L)]
            sv = bs[pl.ds(g * L, L)]
            du = d - lo_v
            m = plsc.bitcast(du, jnp.uint32) < own_u32
            mi = jnp.where(m, ones_v, zeros_v)
            csum = plsc.cumsum(mi)
            pos = jnp.where(m, csum + cnt_v1, trash_v)
            plsc.store_scatter(srcf, [pos], sv)
            plsc.store_scatter(dstf, [pos], du)
            cnt_v1 = cnt_v1 + jnp.take_along_axis(csum, last_v, axis=0)
        return cnt_v1

    def _bload(b, xblk, bsem):
        off = (b % NB) * BW
        pltpu.async_copy(x_hbm.at[pl.ds(off, BW)], xblk, bsem)

    def _bwait(xblk, bsem):
        pltpu.make_async_copy(x_hbm.at[pl.ds(0, BW)], xblk, bsem).wait()

    def _block(b, xblk, ngrp):
        blo_v = jnp.full((L,), b * BR, jnp.int32)

        def _bscan(i, _):
            sf = srcf[pl.ds(i * L, L)]
            su = sf - blo_v
            mb = plsc.bitcast(su, jnp.uint32) < br_u32

            @pl.when(jnp.any(mb))
            def _hit():
                df = dstf[pl.ds(i * L, L)]
                mi = jnp.where(mb, ones_v, zeros_v)
                csum = plsc.cumsum(mi)
                pos = jnp.where(mb, csum - ones_v, mtrash_v)
                plsc.store_scatter(mbs, [pos], su)
                plsc.store_scatter(mbd, [pos], df)
                cntb = jnp.sum(mi)
                suv = mbs[pl.ds(0, L)]
                dfv = mbd[pl.ds(0, L)]

                def _edge(e, _):
                    e_v = jnp.full((L,), e, jnp.int32)
                    su_v = jnp.take_along_axis(suv, e_v, axis=0) * HW
                    d_v = jnp.take_along_axis(dfv, e_v, axis=0)
                    pvs = [plsc.load_gather(xblk, [su_v + pvecs[h]])
                           for h in range(FG // 2)]
                    for h in range(FG // 2):
                        av, bv = plsc.unpack(
                            plsc.bitcast(pvs[h], jnp.bfloat16),
                            format=plsc.PackFormat.INTERLEAVED,
                            preferred_element_type=jnp.float32)
                        plsc.addupdate_scatter(acc, [d_v, favecs[h]], av)
                        plsc.addupdate_scatter(acc, [d_v, fbvecs[h]], bv)
                    return 0
                lax.fori_loop(0, cntb, _edge, 0)
            return 0
        lax.fori_loop(0, ngrp, _bscan, 0)

    bbufs = ((xblk0, bsem0), (xblk1, bsem1))

    def _flush(cnt_v1):
        cnt = jnp.sum(jnp.where(iota_v == 0, cnt_v1, zeros_v)) + 1
        for u in range(2):
            srcf[pl.ds(cnt + u * L, L)] = zeros_v
            dstf[pl.ds(cnt + u * L, L)] = dummy_v
        ngrp = (cnt + L - 1) // L
        _bload(0, *bbufs[0])
        _bload(1, *bbufs[1])

        def _block2(bb, _):
            for p in range(2):
                xblk, bsem = bbufs[p]
                _bwait(xblk, bsem)
                _block(bb * 2 + p, xblk, ngrp)
                _bload(bb * 2 + p + 2, xblk, bsem)
            return 0
        lax.fori_loop(0, NB // 2, _block2, 0)
        _bwait(*bbufs[0])
        _bwait(*bbufs[1])
        return jnp.full((L,), -1, jnp.int32)

    def _eload(t0, b_srcv, b_dstv, b_sem):
        t = (t0 + wid) % NSTRIP
        base_e = t * S
        pltpu.async_copy(src_hbm.at[pl.ds(base_e, S)], b_srcv, b_sem)
        pltpu.async_copy(dst_hbm.at[pl.ds(base_e, S)], b_dstv, b_sem)

    def _ewait(b_srcv, b_dstv, b_sem):
        pltpu.make_async_copy(src_hbm.at[pl.ds(0, S)], b_srcv, b_sem).wait()
        pltpu.make_async_copy(dst_hbm.at[pl.ds(0, S)], b_dstv, b_sem).wait()

    ebufs = ((srcv, dstv, esem0), (srcv1, dstv1, esem1))
    _eload(0, *ebufs[0])
    _eload(1, *ebufs[1])

    def _strip2(tt, cnt_v1):
        for p in range(2):
            bs, bd, bsem = ebufs[p]
            _ewait(bs, bd, bsem)
            cnt_v1 = lax.fori_loop(
                0, S // L // FU,
                functools.partial(_filt_from, bs, bd), cnt_v1)
            _eload(tt * 2 + p + 2, bs, bd, bsem)
        cnt_s = jnp.sum(jnp.where(iota_v == 0, cnt_v1, zeros_v)) + 1
        return lax.cond(cnt_s >= CAPF - 2 * S - 32, _flush,
                        lambda cv: cv, cnt_v1)
    cnt_v1 = lax.fori_loop(0, NSTRIP // 2, _strip2,
                           jnp.full((L,), -1, jnp.int32))
    _ewait(*ebufs[0])
    _ewait(*ebufs[1])
    _flush(cnt_v1)

    @pl.when(wid < NW - 1)
    def _out():
        pltpu.sync_copy(acc.at[pl.ds(0, OWN)], agg_hbm.at[pl.ds(lo, OWN)])

    @pl.when(wid == NW - 1)
    def _out_last():
        pltpu.sync_copy(acc.at[pl.ds(0, N - (NW - 1) * OWN)],
                        agg_hbm.at[pl.ds(lo, N - (NW - 1) * OWN)])


_sc_agg = functools.partial(
    pl.kernel,
    out_type=jax.ShapeDtypeStruct((N, D), jnp.float32),
    mesh=plsc.VectorSubcoreMesh(core_axis_name="c", subcore_axis_name="s",
                                num_cores=NC, num_subcores=NS),
    compiler_params=pltpu.CompilerParams(needs_layout_passes=False),
    scratch_types=[
        pltpu.VMEM((S,), jnp.int32),
        pltpu.VMEM((S,), jnp.int32),
        pltpu.VMEM((S,), jnp.int32),
        pltpu.VMEM((S,), jnp.int32),
        pltpu.VMEM((CAPF,), jnp.int32),
        pltpu.VMEM((CAPF,), jnp.int32),
        pltpu.VMEM((32,), jnp.int32),
        pltpu.VMEM((32,), jnp.int32),
        pltpu.VMEM((BW,), jnp.int32),
        pltpu.VMEM((BW,), jnp.int32),
        pltpu.VMEM((ACC_R, D), jnp.float32),
        pltpu.SemaphoreType.DMA,
        pltpu.SemaphoreType.DMA,
        pltpu.SemaphoreType.DMA,
        pltpu.SemaphoreType.DMA,
    ],
)(_sc_agg_body)


def _mlp_body(x_ref, a_ref, w1_ref, b1_ref, w2_ref, b2_ref, o_ref):
    h = x_ref[...] + a_ref[...]
    h1 = jnp.maximum(jnp.dot(h, w1_ref[...],
                             preferred_element_type=jnp.float32) + b1_ref[...], 0.0)
    o_ref[...] = jnp.maximum(jnp.dot(h1, w2_ref[...],
                                     preferred_element_type=jnp.float32) + b2_ref[...], 0.0)


def _mlp(x, agg, W1, b1, W2, b2):
    BN = 1000
    return pl.pallas_call(
        _mlp_body,
        grid=(N // BN,),
        in_specs=[
            pl.BlockSpec((BN, D), lambda i: (i, 0)),
            pl.BlockSpec((BN, D), lambda i: (i, 0)),
            pl.BlockSpec((D, D), lambda i: (0, 0)),
            pl.BlockSpec((1, D), lambda i: (0, 0)),
            pl.BlockSpec((D, D), lambda i: (0, 0)),
            pl.BlockSpec((1, D), lambda i: (0, 0)),
        ],
        out_specs=pl.BlockSpec((BN, D), lambda i: (i, 0)),
        out_shape=jax.ShapeDtypeStruct((N, D), jnp.float32),
    )(x, agg, W1, b1.reshape(1, D), W2, b2.reshape(1, D))


def kernel(x, edge_index, batch, W1, b1, W2, b2):
    src = edge_index[0]
    dst = edge_index[1]
    xi = lax.bitcast_convert_type(
        x.astype(jnp.bfloat16).reshape(N, D // 2, 2),
        jnp.int32).reshape(N * D // 2)
    agg = _sc_agg(xi, src, dst)
    return _mlp(x, agg, W1, b1, W2, b2)


# speedup vs baseline: 1.3777x; 1.3777x over previous
import functools

import jax
import jax.numpy as jnp
from jax import lax
from jax.experimental import pallas as pl
from jax.experimental.pallas import tpu as pltpu
from jax.experimental.pallas import tpu_sc as plsc

N = 10000
D = 256
E = 160000

NC = 2
NS = 16
NW = NC * NS
L = 16
HW = D // 2

OWN = 320
S = 800
NSTRIP = E // S
FU = 5
CAPF = 8192
ACC_R = 321
DUMMY = OWN
FG = D // L
BR = 100
NB = N // BR
BW = BR * HW


def _sc_agg_body(x_hbm, src_hbm, dst_hbm, agg_hbm,
                 srcv, dstv, srcv1, dstv1, srcf, dstf, mbs, mbd,
                 xblk0, xblk1, acc, bsem0, bsem1, esem0, esem1):
    c = lax.axis_index("c")
    s = lax.axis_index("s")
    wid = c * NS + s
    lo = wid * OWN
    lo_v = jnp.full((L,), lo, jnp.int32)
    own_u32 = jnp.full((L,), OWN, jnp.uint32)
    br_u32 = jnp.full((L,), BR, jnp.uint32)
    ones_v = jnp.ones((L,), jnp.int32)
    zeros_v = jnp.zeros((L,), jnp.int32)
    trash_v = jnp.full((L,), CAPF - 1, jnp.int32)
    mtrash_v = jnp.full((L,), 31, jnp.int32)
    dummy_v = jnp.full((L,), DUMMY, jnp.int32)
    iota_v = lax.iota(jnp.int32, L)
    last_v = jnp.full((L,), L - 1, jnp.int32)
    zero16f = jnp.zeros((L,), jnp.float32)
    pvecs = [iota_v + h * L for h in range(FG // 2)]
    favecs = [iota_v * 2 + h * 2 * L for h in range(FG // 2)]
    fbvecs = [favecs[h] + 1 for h in range(FG // 2)]

    def _zacc(i, _):
        plsc.store_scatter(acc, [jnp.full((L,), i // FG, jnp.int32),
                                 iota_v + (i % FG) * L], zero16f)
        return 0
    lax.fori_loop(0, ACC_R * FG, _zacc, 0)

    def _filt_from(bs, bd, i, cnt_v1):
        for u in range(FU):
            g = i * FU + u
            d = bd[pl.ds(g * L, L)]
            sv = bs[pl.ds(g * L, L)]
            du = d - lo_v
            m = plsc.bitcast(du, jnp.uint32) < own_u32
            mi = jnp.where(m, ones_v, zeros_v)
            csum = plsc.cumsum(mi)
            pos = jnp.where(m, csum + cnt_v1, trash_v)
            plsc.store_scatter(srcf, [pos], sv)
            plsc.store_scatter(dstf, [pos], du)
            cnt_v1 = cnt_v1 + jnp.take_along_axis(csum, last_v, axis=0)
        return cnt_v1

    def _bload(b, xblk, bsem):
        off = (b % NB) * BW
        pltpu.async_copy(x_hbm.at[pl.ds(off, BW)], xblk, bsem)

    def _bwait(xblk, bsem):
        pltpu.make_async_copy(x_hbm.at[pl.ds(0, BW)], xblk, bsem).wait()

    def _block(b, xblk, ngrp):
        blo_v = jnp.full((L,), b * BR, jnp.int32)

        def _bscan(i, _):
            sf = srcf[pl.ds(i * L, L)]
            df = dstf[pl.ds(i * L, L)]
            su = sf - blo_v
            mb = plsc.bitcast(su, jnp.uint32) < br_u32
            mi = jnp.where(mb, ones_v, zeros_v)
            csum = plsc.cumsum(mi)
            pos = jnp.where(mb, csum - ones_v, mtrash_v)
            plsc.store_scatter(mbs, [pos], su)
            plsc.store_scatter(mbd, [pos], df)
            cntb = jnp.sum(mi)
            suv = mbs[pl.ds(0, L)]
            dfv = mbd[pl.ds(0, L)]

            def _edge(e, _):
                e_v = jnp.full((L,), e, jnp.int32)
                su_v = jnp.take_along_axis(suv, e_v, axis=0) * HW
                d_v = jnp.take_along_axis(dfv, e_v, axis=0)
                pvs = [plsc.load_gather(xblk, [su_v + pvecs[h]])
                       for h in range(FG // 2)]
                for h in range(FG // 2):
                    av, bv = plsc.unpack(
                        plsc.bitcast(pvs[h], jnp.bfloat16),
                        format=plsc.PackFormat.INTERLEAVED,
                        preferred_element_type=jnp.float32)
                    plsc.addupdate_scatter(acc, [d_v, favecs[h]], av)
                    plsc.addupdate_scatter(acc, [d_v, fbvecs[h]], bv)
                return 0
            lax.fori_loop(0, cntb, _edge, 0)
            return 0
        lax.fori_loop(0, ngrp, _bscan, 0)

    bbufs = ((xblk0, bsem0), (xblk1, bsem1))

    def _flush(cnt_v1):
        cnt = jnp.sum(jnp.where(iota_v == 0, cnt_v1, zeros_v)) + 1
        for u in range(2):
            srcf[pl.ds(cnt + u * L, L)] = zeros_v
            dstf[pl.ds(cnt + u * L, L)] = dummy_v
        ngrp = (cnt + L - 1) // L
        _bload(0, *bbufs[0])
        _bload(1, *bbufs[1])

        def _block2(bb, _):
            for p in range(2):
                xblk, bsem = bbufs[p]
                _bwait(xblk, bsem)
                _block(bb * 2 + p, xblk, ngrp)
                _bload(bb * 2 + p + 2, xblk, bsem)
            return 0
        lax.fori_loop(0, NB // 2, _block2, 0)
        _bwait(*bbufs[0])
        _bwait(*bbufs[1])
        return jnp.full((L,), -1, jnp.int32)

    def _eload(t0, b_srcv, b_dstv, b_sem):
        t = (t0 + wid) % NSTRIP
        base_e = t * S
        pltpu.async_copy(src_hbm.at[pl.ds(base_e, S)], b_srcv, b_sem)
        pltpu.async_copy(dst_hbm.at[pl.ds(base_e, S)], b_dstv, b_sem)

    def _ewait(b_srcv, b_dstv, b_sem):
        pltpu.make_async_copy(src_hbm.at[pl.ds(0, S)], b_srcv, b_sem).wait()
        pltpu.make_async_copy(dst_hbm.at[pl.ds(0, S)], b_dstv, b_sem).wait()

    ebufs = ((srcv, dstv, esem0), (srcv1, dstv1, esem1))
    _eload(0, *ebufs[0])
    _eload(1, *ebufs[1])

    def _strip2(tt, cnt_v1):
        for p in range(2):
            bs, bd, bsem = ebufs[p]
            _ewait(bs, bd, bsem)
            cnt_v1 = lax.fori_loop(
                0, S // L // FU,
                functools.partial(_filt_from, bs, bd), cnt_v1)
            _eload(tt * 2 + p + 2, bs, bd, bsem)
        cnt_s = jnp.sum(jnp.where(iota_v == 0, cnt_v1, zeros_v)) + 1
        return lax.cond(cnt_s >= CAPF - 2 * S - 32, _flush,
                        lambda cv: cv, cnt_v1)
    cnt_v1 = lax.fori_loop(0, NSTRIP // 2, _strip2,
                           jnp.full((L,), -1, jnp.int32))
    _ewait(*ebufs[0])
    _ewait(*ebufs[1])
    _flush(cnt_v1)

    @pl.when(wid < NW - 1)
    def _out():
        pltpu.sync_copy(acc.at[pl.ds(0, OWN)], agg_hbm.at[pl.ds(lo, OWN)])

    @pl.when(wid == NW - 1)
    def _out_last():
        pltpu.sync_copy(acc.at[pl.ds(0, N - (NW - 1) * OWN)],
                        agg_hbm.at[pl.ds(lo, N - (NW - 1) * OWN)])


_sc_agg = functools.partial(
    pl.kernel,
    out_type=jax.ShapeDtypeStruct((N, D), jnp.float32),
    mesh=plsc.VectorSubcoreMesh(core_axis_name="c", subcore_axis_name="s",
                                num_cores=NC, num_subcores=NS),
    compiler_params=pltpu.CompilerParams(needs_layout_passes=False),
    scratch_types=[
        pltpu.VMEM((S,), jnp.int32),
        pltpu.VMEM((S,), jnp.int32),
        pltpu.VMEM((S,), jnp.int32),
        pltpu.VMEM((S,), jnp.int32),
        pltpu.VMEM((CAPF,), jnp.int32),
        pltpu.VMEM((CAPF,), jnp.int32),
        pltpu.VMEM((32,), jnp.int32),
        pltpu.VMEM((32,), jnp.int32),
        pltpu.VMEM((BW,), jnp.int32),
        pltpu.VMEM((BW,), jnp.int32),
        pltpu.VMEM((ACC_R, D), jnp.float32),
        pltpu.SemaphoreType.DMA,
        pltpu.SemaphoreType.DMA,
        pltpu.SemaphoreType.DMA,
        pltpu.SemaphoreType.DMA,
    ],
)(_sc_agg_body)


def _mlp_body(x_ref, a_ref, w1_ref, b1_ref, w2_ref, b2_ref, o_ref):
    h = x_ref[...] + a_ref[...]
    h1 = jnp.maximum(jnp.dot(h, w1_ref[...],
                             preferred_element_type=jnp.float32) + b1_ref[...], 0.0)
    o_ref[...] = jnp.maximum(jnp.dot(h1, w2_ref[...],
                                     preferred_element_type=jnp.float32) + b2_ref[...], 0.0)


def _mlp(x, agg, W1, b1, W2, b2):
    BN = 1000
    return pl.pallas_call(
        _mlp_body,
        grid=(N // BN,),
        in_specs=[
            pl.BlockSpec((BN, D), lambda i: (i, 0)),
            pl.BlockSpec((BN, D), lambda i: (i, 0)),
            pl.BlockSpec((D, D), lambda i: (0, 0)),
            pl.BlockSpec((1, D), lambda i: (0, 0)),
            pl.BlockSpec((D, D), lambda i: (0, 0)),
            pl.BlockSpec((1, D), lambda i: (0, 0)),
        ],
        out_specs=pl.BlockSpec((BN, D), lambda i: (i, 0)),
        out_shape=jax.ShapeDtypeStruct((N, D), jnp.float32),
    )(x, agg, W1, b1.reshape(1, D), W2, b2.reshape(1, D))


def kernel(x, edge_index, batch, W1, b1, W2, b2):
    src = edge_index[0]
    dst = edge_index[1]
    xi = lax.bitcast_convert_type(
        x.astype(jnp.bfloat16).reshape(N, D // 2, 2),
        jnp.int32).reshape(N * D // 2)
    agg = _sc_agg(xi, src, dst)
    return _mlp(x, agg, W1, b1, W2, b2)


# speedup vs baseline: 1.8854x; 1.3686x over previous
import functools

import jax
import jax.numpy as jnp
from jax import lax
from jax.experimental import pallas as pl
from jax.experimental.pallas import tpu as pltpu
from jax.experimental.pallas import tpu_sc as plsc

N = 10000
D = 256
E = 160000

NC = 2
NS = 16
NW = NC * NS
L = 16
HW = D // 2

OWN = 320
S = 800
NSTRIP = E // S
FU = 5
CAPF = 8192
ACC_R = 321
DUMMY = OWN
FG = D // L
BR = 200
NB = N // BR
BW = BR * HW


def _sc_agg_body(x_hbm, src_hbm, dst_hbm, agg_hbm,
                 srcv, dstv, srcv1, dstv1, srcf, dstf, mbs, mbd,
                 xblk0, acc, esem0, esem1):
    c = lax.axis_index("c")
    s = lax.axis_index("s")
    wid = c * NS + s
    lo = wid * OWN
    lo_v = jnp.full((L,), lo, jnp.int32)
    own_u32 = jnp.full((L,), OWN, jnp.uint32)
    br_u32 = jnp.full((L,), BR, jnp.uint32)
    ones_v = jnp.ones((L,), jnp.int32)
    zeros_v = jnp.zeros((L,), jnp.int32)
    trash_v = jnp.full((L,), CAPF - 1, jnp.int32)
    mtrash_v = jnp.full((L,), 31, jnp.int32)
    dummy_v = jnp.full((L,), DUMMY, jnp.int32)
    iota_v = lax.iota(jnp.int32, L)
    last_v = jnp.full((L,), L - 1, jnp.int32)
    zero16f = jnp.zeros((L,), jnp.float32)
    pvecs = [iota_v + h * L for h in range(FG // 2)]
    favecs = [iota_v * 2 + h * 2 * L for h in range(FG // 2)]
    fbvecs = [favecs[h] + 1 for h in range(FG // 2)]

    def _zacc(i, _):
        plsc.store_scatter(acc, [jnp.full((L,), i // FG, jnp.int32),
                                 iota_v + (i % FG) * L], zero16f)
        return 0
    lax.fori_loop(0, ACC_R * FG, _zacc, 0)

    def _filt_from(bs, bd, i, cnt_v1):
        for u in range(FU):
            g = i * FU + u
            d = bd[pl.ds(g * L, L)]
            sv = bs[pl.ds(g * L, L)]
            du = d - lo_v
            m = plsc.bitcast(du, jnp.uint32) < own_u32
            mi = jnp.where(m, ones_v, zeros_v)
            csum = plsc.cumsum(mi)
            pos = jnp.where(m, csum + cnt_v1, trash_v)
            plsc.store_scatter(srcf, [pos], sv)
            plsc.store_scatter(dstf, [pos], du)
            cnt_v1 = cnt_v1 + jnp.take_along_axis(csum, last_v, axis=0)
        return cnt_v1

    def _block(b, xblk, ngrp):
        blo_v = jnp.full((L,), b * BR, jnp.int32)

        def _bscan(i, _):
            sf = srcf[pl.ds(i * L, L)]
            df = dstf[pl.ds(i * L, L)]
            su = sf - blo_v
            mb = plsc.bitcast(su, jnp.uint32) < br_u32
            mi = jnp.where(mb, ones_v, zeros_v)
            csum = plsc.cumsum(mi)
            pos = jnp.where(mb, csum - ones_v, mtrash_v)
            plsc.store_scatter(mbs, [pos], su)
            plsc.store_scatter(mbd, [pos], df)
            cntb = jnp.sum(mi)
            suv = mbs[pl.ds(0, L)]
            dfv = mbd[pl.ds(0, L)]

            def _edge(e, _):
                e_v = jnp.full((L,), e, jnp.int32)
                su_v = jnp.take_along_axis(suv, e_v, axis=0) * HW
                d_v = jnp.take_along_axis(dfv, e_v, axis=0)
                pvs = [plsc.load_gather(xblk, [su_v + pvecs[h]])
                       for h in range(FG // 2)]
                for h in range(FG // 2):
                    av, bv = plsc.unpack(
                        plsc.bitcast(pvs[h], jnp.bfloat16),
                        format=plsc.PackFormat.INTERLEAVED,
                        preferred_element_type=jnp.float32)
                    plsc.addupdate_scatter(acc, [d_v, favecs[h]], av)
                    plsc.addupdate_scatter(acc, [d_v, fbvecs[h]], bv)
                return 0
            lax.fori_loop(0, cntb, _edge, 0)
            return 0
        lax.fori_loop(0, ngrp, _bscan, 0)

    def _flush(cnt_v1):
        cnt = jnp.sum(jnp.where(iota_v == 0, cnt_v1, zeros_v)) + 1
        for u in range(2):
            srcf[pl.ds(cnt + u * L, L)] = zeros_v
            dstf[pl.ds(cnt + u * L, L)] = dummy_v
        ngrp = (cnt + L - 1) // L

        def _block1(b, _):
            pltpu.sync_copy(x_hbm.at[pl.ds(b * BW, BW)], xblk0)
            _block(b, xblk0, ngrp)
            return 0
        lax.fori_loop(0, NB, _block1, 0)
        return jnp.full((L,), -1, jnp.int32)

    def _eload(t0, b_srcv, b_dstv, b_sem):
        t = (t0 + wid) % NSTRIP
        base_e = t * S
        pltpu.async_copy(src_hbm.at[pl.ds(base_e, S)], b_srcv, b_sem)
        pltpu.async_copy(dst_hbm.at[pl.ds(base_e, S)], b_dstv, b_sem)

    def _ewait(b_srcv, b_dstv, b_sem):
        pltpu.make_async_copy(src_hbm.at[pl.ds(0, S)], b_srcv, b_sem).wait()
        pltpu.make_async_copy(dst_hbm.at[pl.ds(0, S)], b_dstv, b_sem).wait()

    ebufs = ((srcv, dstv, esem0), (srcv1, dstv1, esem1))
    _eload(0, *ebufs[0])
    _eload(1, *ebufs[1])

    def _strip2(tt, cnt_v1):
        for p in range(2):
            bs, bd, bsem = ebufs[p]
            _ewait(bs, bd, bsem)
            cnt_v1 = lax.fori_loop(
                0, S // L // FU,
                functools.partial(_filt_from, bs, bd), cnt_v1)
            _eload(tt * 2 + p + 2, bs, bd, bsem)
        cnt_s = jnp.sum(jnp.where(iota_v == 0, cnt_v1, zeros_v)) + 1
        return lax.cond(cnt_s >= CAPF - 2 * S - 32, _flush,
                        lambda cv: cv, cnt_v1)
    cnt_v1 = lax.fori_loop(0, NSTRIP // 2, _strip2,
                           jnp.full((L,), -1, jnp.int32))
    _ewait(*ebufs[0])
    _ewait(*ebufs[1])
    _flush(cnt_v1)

    @pl.when(wid < NW - 1)
    def _out():
        pltpu.sync_copy(acc.at[pl.ds(0, OWN)], agg_hbm.at[pl.ds(lo, OWN)])

    @pl.when(wid == NW - 1)
    def _out_last():
        pltpu.sync_copy(acc.at[pl.ds(0, N - (NW - 1) * OWN)],
                        agg_hbm.at[pl.ds(lo, N - (NW - 1) * OWN)])


_sc_agg = functools.partial(
    pl.kernel,
    out_type=jax.ShapeDtypeStruct((N, D), jnp.float32),
    mesh=plsc.VectorSubcoreMesh(core_axis_name="c", subcore_axis_name="s",
                                num_cores=NC, num_subcores=NS),
    compiler_params=pltpu.CompilerParams(needs_layout_passes=False),
    scratch_types=[
        pltpu.VMEM((S,), jnp.int32),
        pltpu.VMEM((S,), jnp.int32),
        pltpu.VMEM((S,), jnp.int32),
        pltpu.VMEM((S,), jnp.int32),
        pltpu.VMEM((CAPF,), jnp.int32),
        pltpu.VMEM((CAPF,), jnp.int32),
        pltpu.VMEM((32,), jnp.int32),
        pltpu.VMEM((32,), jnp.int32),
        pltpu.VMEM((BW,), jnp.int32),
        pltpu.VMEM((ACC_R, D), jnp.float32),
        pltpu.SemaphoreType.DMA,
        pltpu.SemaphoreType.DMA,
    ],
)(_sc_agg_body)


def _mlp_body(x_ref, a_ref, w1_ref, b1_ref, w2_ref, b2_ref, o_ref):
    h = x_ref[...] + a_ref[...]
    h1 = jnp.maximum(jnp.dot(h, w1_ref[...],
                             preferred_element_type=jnp.float32) + b1_ref[...], 0.0)
    o_ref[...] = jnp.maximum(jnp.dot(h1, w2_ref[...],
                                     preferred_element_type=jnp.float32) + b2_ref[...], 0.0)


def _mlp(x, agg, W1, b1, W2, b2):
    BN = 1000
    return pl.pallas_call(
        _mlp_body,
        grid=(N // BN,),
        in_specs=[
            pl.BlockSpec((BN, D), lambda i: (i, 0)),
            pl.BlockSpec((BN, D), lambda i: (i, 0)),
            pl.BlockSpec((D, D), lambda i: (0, 0)),
            pl.BlockSpec((1, D), lambda i: (0, 0)),
            pl.BlockSpec((D, D), lambda i: (0, 0)),
            pl.BlockSpec((1, D), lambda i: (0, 0)),
        ],
        out_specs=pl.BlockSpec((BN, D), lambda i: (i, 0)),
        out_shape=jax.ShapeDtypeStruct((N, D), jnp.float32),
    )(x, agg, W1, b1.reshape(1, D), W2, b2.reshape(1, D))


def kernel(x, edge_index, batch, W1, b1, W2, b2):
    src = edge_index[0]
    dst = edge_index[1]
    xi = lax.bitcast_convert_type(
        x.astype(jnp.bfloat16).reshape(N, D // 2, 2),
        jnp.int32).reshape(N * D // 2)
    agg = _sc_agg(xi, src, dst)
    return _mlp(x, agg, W1, b1, W2, b2)
